# baseline (speedup 1.0000x reference)
import functools

import jax
import jax.numpy as jnp
from jax import lax
from jax.experimental import pallas as pl
from jax.experimental.pallas import tpu as pltpu
from jax.experimental.pallas import tpu_sc as plsc

_NC = 2
_NS = 16
_NW = _NC * _NS

_CHUNK = 128
_K = 4
_BLK = 2048


def _sc_gather_bf16(table, idx3):
    nw, j_steps, chunk = idx3.shape
    rows_out = nw * j_steps * chunk
    feat = table.shape[1]
    per_w = j_steps * chunk
    mesh = plsc.VectorSubcoreMesh(core_axis_name="c", subcore_axis_name="s")

    @functools.partial(
        pl.kernel,
        mesh=mesh,
        out_type=jax.ShapeDtypeStruct((rows_out // 2, feat), jnp.uint32),
        scratch_types=(
            [pltpu.VMEM((j_steps, chunk), jnp.int32),
             pltpu.VMEM((2 * chunk, feat), jnp.float32),
             pltpu.VMEM((per_w // 2, feat), jnp.uint32)]
            + [pltpu.SemaphoreType.DMA] * 2
            + [pltpu.SemaphoreType.DMA]
        ),
    )
    def gather_k(table_hbm, idx_hbm, out_hbm, idx_v, rows_v, ebf_v, *sems):
        gsems, ssem = sems[:2], sems[2]
        wid = lax.axis_index("s") * _NC + lax.axis_index("c")
        pltpu.sync_copy(idx_hbm.at[wid], idx_v)
        base2 = wid * (per_w // 2)

        def fire(j):
            return pltpu.async_copy(
                table_hbm.at[idx_v.at[j]],
                rows_v.at[pl.ds((j % 2) * chunk, chunk)],
                gsems[j % 2],
            )

        gathers = [fire(j) for j in range(min(2, j_steps))]
        half = jnp.uint32(0x8000)
        himask = jnp.uint32(0xFFFF0000)
        scatters = []
        for j in range(j_steps):
            gathers[j].wait()
            slot = (j % 2) * chunk

            @plsc.parallel_loop(0, chunk // 2, 1, unroll=4)
            def conv_pair(r2, j=j, slot=slot):
                for s in range(feat // 16):
                    a = rows_v[slot + 2 * r2, pl.ds(16 * s, 16)]
                    b = rows_v[slot + 2 * r2 + 1, pl.ds(16 * s, 16)]
                    au = lax.bitcast_convert_type(a, jnp.uint32)
                    bu = lax.bitcast_convert_type(b, jnp.uint32)
                    lo = lax.shift_right_logical(au + half, jnp.uint32(16))
                    hi = (bu + half) & himask
                    ebf_v[j * (chunk // 2) + r2, pl.ds(16 * s, 16)] = lo | hi

            if j + 2 < j_steps:
                gathers.append(fire(j + 2))
            scatters.append(
                pltpu.async_copy(
                    ebf_v.at[pl.ds(j * (chunk // 2), chunk // 2)],
                    out_hbm.at[pl.ds(base2 + j * (chunk // 2), chunk // 2)],
                    ssem,
                )
            )
        for s in scatters:
            s.wait()

    return gather_k(table, idx3)


def _film_body(e_ref, x_ref, w_ref, b_ref, out_ref):
    feat = e_ref.shape[-1]
    eu = e_ref[...]
    ea = lax.bitcast_convert_type(eu << jnp.uint32(16), jnp.float32)
    eb = lax.bitcast_convert_type(eu & jnp.uint32(0xFFFF0000), jnp.float32)
    w = w_ref[...]
    bb = b_ref[...]
    he = jnp.dot(ea.astype(jnp.bfloat16), w,
                 preferred_element_type=jnp.float32) + bb
    ho = jnp.dot(eb.astype(jnp.bfloat16), w,
                 preferred_element_type=jnp.float32) + bb
    xx = x_ref[...]
    oute = xx[:, :feat] * he[:, :feat] + he[:, feat:]
    outo = xx[:, feat:] * ho[:, :feat] + ho[:, feat:]
    out_ref[...] = jnp.concatenate([oute, outo], axis=1)


def _film_body_chained(e_ref, x_ref, w_ref, b_ref, buf_ref, out_ref):
    del buf_ref
    _film_body(e_ref, x_ref, w_ref, b_ref, out_ref)


def _film_chunk(e_k, x2, W16, b2, buf, k, rows, feat):
    pairs = e_k.shape[0]
    blk2 = _BLK // 2
    nb = pairs // blk2
    e_spec = pl.BlockSpec((blk2, feat), lambda i: (i, 0))
    x_spec = pl.BlockSpec((blk2, 2 * feat), lambda i: (k * nb + i, 0))
    w_spec = pl.BlockSpec((feat, 2 * feat), lambda i: (0, 0))
    b_spec = pl.BlockSpec((1, 2 * feat), lambda i: (0, 0))
    out_spec = pl.BlockSpec((blk2, 2 * feat), lambda i: (k * nb + i, 0))
    out_shape = jax.ShapeDtypeStruct((rows // 2, 2 * feat), jnp.float32)
    if buf is None:
        return pl.pallas_call(
            _film_body,
            grid=(nb,),
            in_specs=[e_spec, x_spec, w_spec, b_spec],
            out_specs=out_spec,
            out_shape=out_shape,
        )(e_k, x2, W16, b2)
    buf_spec = pl.BlockSpec((8, 2 * feat), lambda i: (0, 0))
    return pl.pallas_call(
        _film_body_chained,
        grid=(nb,),
        in_specs=[e_spec, x_spec, w_spec, b_spec, buf_spec],
        out_specs=out_spec,
        out_shape=out_shape,
        input_output_aliases={4: 0},
    )(e_k, x2, W16, b2, buf)


def kernel(x, patch_idx, group_idx, embeddings, W, b):
    batch, patch, feat = x.shape
    n_groups, n_nodes, _ = embeddings.shape
    rows = batch * patch

    table = embeddings.reshape(n_groups * n_nodes, feat)
    flat_idx = (group_idx.astype(jnp.int32)[:, None] * n_nodes
                + patch_idx.astype(jnp.int32))
    j_steps = rows // (_K * _NW * _CHUNK)
    idx4 = flat_idx.reshape(_K, _NW, j_steps, _CHUNK)

    e_chunks = [_sc_gather_bf16(table, idx4[k]) for k in range(_K)]

    W16 = W.astype(jnp.bfloat16)
    x2 = x.reshape(rows // 2, 2 * feat)
    b2 = b.reshape(1, 2 * feat)
    buf = None
    for k in range(_K):
        buf = _film_chunk(e_chunks[k], x2, W16, b2, buf, k, rows, feat)
    return buf.reshape(batch, patch, feat)


# speedup vs baseline: 1.7618x; 1.7618x over previous
import functools

import jax
import jax.numpy as jnp
from jax import lax
from jax.experimental import pallas as pl
from jax.experimental.pallas import tpu as pltpu
from jax.experimental.pallas import tpu_sc as plsc

_NC = 2
_NS = 16
_NW = _NC * _NS

_CHUNK = 128
_K = 4
_BLK = 2048
_HALF = _BLK // 2


def _sc_gather_pack(table, idx4):
    nw, j_steps, _, chunk = idx4.shape
    pairs_out = nw * j_steps * chunk
    feat = table.shape[1]
    per_w = j_steps * chunk
    mesh = plsc.VectorSubcoreMesh(core_axis_name="c", subcore_axis_name="s")

    @functools.partial(
        pl.kernel,
        mesh=mesh,
        out_type=jax.ShapeDtypeStruct((pairs_out, feat), jnp.uint32),
        scratch_types=(
            [pltpu.VMEM((j_steps, 2, chunk), jnp.int32),
             pltpu.VMEM((2 * per_w, feat), jnp.float32),
             pltpu.VMEM((per_w, feat), jnp.uint32)]
            + [pltpu.SemaphoreType.DMA] * (2 * j_steps)
            + [pltpu.SemaphoreType.DMA]
        ),
    )
    def gather_k(table_hbm, idx_hbm, out_hbm, idx_v, rows_v, ebf_v, *sems):
        gsems, ssem = sems[:2 * j_steps], sems[2 * j_steps]
        wid = lax.axis_index("s") * _NC + lax.axis_index("c")
        pltpu.sync_copy(idx_hbm.at[wid], idx_v)
        base = wid * per_w
        gathers = [
            pltpu.async_copy(
                table_hbm.at[idx_v.at[j, par]],
                rows_v.at[pl.ds((2 * j + par) * chunk, chunk)],
                gsems[2 * j + par],
            )
            for j in range(j_steps)
            for par in range(2)
        ]
        half = jnp.uint32(0x8000)
        himask = jnp.uint32(0xFFFF0000)
        scatters = []
        for j in range(j_steps):
            gathers[2 * j].wait()
            gathers[2 * j + 1].wait()
            lo_slot = 2 * j * chunk
            hi_slot = lo_slot + chunk

            @plsc.parallel_loop(0, chunk, 1, unroll=4)
            def conv_pair(r, j=j, lo_slot=lo_slot, hi_slot=hi_slot):
                for s in range(feat // 16):
                    a = rows_v[lo_slot + r, pl.ds(16 * s, 16)]
                    b = rows_v[hi_slot + r, pl.ds(16 * s, 16)]
                    au = lax.bitcast_convert_type(a, jnp.uint32)
                    bu = lax.bitcast_convert_type(b, jnp.uint32)
                    lo = lax.shift_right_logical(au + half, jnp.uint32(16))
                    hi = (bu + half) & himask
                    ebf_v[j * chunk + r, pl.ds(16 * s, 16)] = lo | hi

            scatters.append(
                pltpu.async_copy(
                    ebf_v.at[pl.ds(j * chunk, chunk)],
                    out_hbm.at[pl.ds(base + j * chunk, chunk)],
                    ssem,
                )
            )
        for s in scatters:
            s.wait()

    return gather_k(table, idx4)


def _film_body(e_ref, x_ref, w_ref, b_ref, out_ref):
    feat = e_ref.shape[-1]
    eu = e_ref[...]
    ea = lax.bitcast_convert_type(eu << jnp.uint32(16), jnp.float32)
    eb = lax.bitcast_convert_type(eu & jnp.uint32(0xFFFF0000), jnp.float32)
    w = w_ref[...]
    bb = b_ref[...]
    ha = jnp.dot(ea.astype(jnp.bfloat16), w,
                 preferred_element_type=jnp.float32) + bb
    hb = jnp.dot(eb.astype(jnp.bfloat16), w,
                 preferred_element_type=jnp.float32) + bb
    xx = x_ref[...]
    outa = xx[:_HALF] * ha[:, :feat] + ha[:, feat:]
    outb = xx[_HALF:] * hb[:, :feat] + hb[:, feat:]
    out_ref[...] = jnp.concatenate([outa, outb], axis=0)


def _film_body_chained(e_ref, x_ref, w_ref, b_ref, buf_ref, out_ref):
    del buf_ref
    _film_body(e_ref, x_ref, w_ref, b_ref, out_ref)


def _film_chunk(e_k, x2, W16, b2, buf, k, rows, feat):
    pairs = e_k.shape[0]
    nb = pairs // _HALF
    e_spec = pl.BlockSpec((_HALF, feat), lambda i: (i, 0))
    x_spec = pl.BlockSpec((_BLK, feat), lambda i: (k * nb + i, 0))
    w_spec = pl.BlockSpec((feat, 2 * feat), lambda i: (0, 0))
    b_spec = pl.BlockSpec((1, 2 * feat), lambda i: (0, 0))
    out_spec = pl.BlockSpec((_BLK, feat), lambda i: (k * nb + i, 0))
    out_shape = jax.ShapeDtypeStruct((rows, feat), jnp.float32)
    if buf is None:
        return pl.pallas_call(
            _film_body,
            grid=(nb,),
            in_specs=[e_spec, x_spec, w_spec, b_spec],
            out_specs=out_spec,
            out_shape=out_shape,
        )(e_k, x2, W16, b2)
    buf_spec = pl.BlockSpec((8, feat), lambda i: (0, 0))
    return pl.pallas_call(
        _film_body_chained,
        grid=(nb,),
        in_specs=[e_spec, x_spec, w_spec, b_spec, buf_spec],
        out_specs=out_spec,
        out_shape=out_shape,
        input_output_aliases={4: 0},
    )(e_k, x2, W16, b2, buf)


def kernel(x, patch_idx, group_idx, embeddings, W, b):
    batch, patch, feat = x.shape
    n_groups, n_nodes, _ = embeddings.shape
    rows = batch * patch

    table = embeddings.reshape(n_groups * n_nodes, feat)
    flat_idx = (group_idx.astype(jnp.int32)[:, None] * n_nodes
                + patch_idx.astype(jnp.int32))
    flat2 = flat_idx.reshape(rows // _BLK, 2, _HALF)
    j_steps = rows // (2 * _K * _NW * _CHUNK)
    ev = flat2[:, 0, :].reshape(_K, _NW, j_steps, 1, _CHUNK)
    od = flat2[:, 1, :].reshape(_K, _NW, j_steps, 1, _CHUNK)
    idx5 = jnp.concatenate([ev, od], axis=3)

    e_chunks = [_sc_gather_pack(table, idx5[k]) for k in range(_K)]

    W16 = W.astype(jnp.bfloat16)
    x2 = x.reshape(rows, feat)
    b2 = b.reshape(1, 2 * feat)
    buf = None
    for k in range(_K):
        buf = _film_chunk(e_chunks[k], x2, W16, b2, buf, k, rows, feat)
    return buf.reshape(batch, patch, feat)
